# baseline (speedup 1.0000x reference)
import functools

import jax
import jax.numpy as jnp
from jax import lax
from jax.experimental import pallas as pl
from jax.experimental.pallas import tpu as pltpu
from jax.experimental.pallas import tpu_sc as plsc

D_MODEL = 64
NUM_CORES = 2
NUM_SUBCORES = 16
NUM_WORKERS = NUM_CORES * NUM_SUBCORES
CHUNK = 256
LANES = 16
NBUF = 2


def _sc_body(x_hbm, ts_hbm, tab_hbm, out_hbm, idx_v, rows_v, x_v, res_v,
             gsem0, gsem1, xsem0, xsem1, osem0, osem1):
    gsem = (gsem0, gsem1)
    xsem = (xsem0, xsem1)
    osem = (osem0, osem1)
    n_tok = x_hbm.shape[0]
    per_w = n_tok // NUM_WORKERS
    n_chunks = per_w // CHUNK
    wid = lax.axis_index("s") * NUM_CORES + lax.axis_index("c")
    base_w = wid * per_w

    def issue(ci, b):
        base = base_w + ci * CHUNK
        pltpu.sync_copy(ts_hbm.at[pl.ds(base, CHUNK)], idx_v.at[b])
        pltpu.async_copy(tab_hbm.at[idx_v.at[b]], rows_v.at[b], gsem[b])
        pltpu.async_copy(x_hbm.at[pl.ds(base, CHUNK)], x_v.at[b], xsem[b])

    for b in range(NBUF):
        issue(b, b)

    @pl.loop(0, n_chunks, step=NBUF)
    def _(g):
        for b in range(NBUF):
            ci = g + b
            pltpu.make_async_copy(
                tab_hbm.at[idx_v.at[b]], rows_v.at[b], gsem[b]).wait()
            pltpu.make_async_copy(
                x_hbm.at[pl.ds(0, CHUNK)], x_v.at[b], xsem[b]).wait()

            @pl.when(ci >= NBUF)
            def _():
                pltpu.make_async_copy(
                    res_v.at[b], out_hbm.at[pl.ds(0, CHUNK)], osem[b]).wait()

            @plsc.parallel_loop(0, CHUNK, unroll=8)
            def _(i):
                for k in range(D_MODEL // LANES):
                    sl = pl.ds(k * LANES, LANES)
                    res_v[b, i, sl] = x_v[b, i, sl] + rows_v[b, i, sl]

            base = base_w + ci * CHUNK
            pltpu.async_copy(res_v.at[b], out_hbm.at[pl.ds(base, CHUNK)],
                             osem[b])

            @pl.when(ci + NBUF < n_chunks)
            def _():
                issue(ci + NBUF, b)

    for b in range(NBUF):
        pltpu.make_async_copy(
            res_v.at[b], out_hbm.at[pl.ds(0, CHUNK)], osem[b]).wait()


@jax.jit
def _sc_call(xf, ts, table):
    n_tok = xf.shape[0]
    mesh = plsc.VectorSubcoreMesh(core_axis_name="c", subcore_axis_name="s")
    f = pl.kernel(
        _sc_body,
        mesh=mesh,
        out_type=jax.ShapeDtypeStruct((n_tok, D_MODEL), jnp.float32),
        scratch_types=[
            pltpu.VMEM((NBUF, CHUNK), jnp.int32),
            pltpu.VMEM((NBUF, CHUNK, D_MODEL), jnp.float32),
            pltpu.VMEM((NBUF, CHUNK, D_MODEL), jnp.float32),
            pltpu.VMEM((NBUF, CHUNK, D_MODEL), jnp.float32),
            pltpu.SemaphoreType.DMA,
            pltpu.SemaphoreType.DMA,
            pltpu.SemaphoreType.DMA,
            pltpu.SemaphoreType.DMA,
            pltpu.SemaphoreType.DMA,
            pltpu.SemaphoreType.DMA,
        ],
        compiler_params=pltpu.CompilerParams(use_tc_tiling_on_sc=False),
    )
    return f(xf, ts, table)


def kernel(x, timestamps, time_embedding):
    b, s, d = x.shape
    xf = x.reshape(b * s, d)
    ts = timestamps.reshape(-1).astype(jnp.int32)
    out = _sc_call(xf, ts, time_embedding)
    return out.reshape(b, s, d)


# speedup vs baseline: 1.3946x; 1.3946x over previous
import functools

import jax
import jax.numpy as jnp
from jax import lax
from jax.experimental import pallas as pl
from jax.experimental.pallas import tpu as pltpu
from jax.experimental.pallas import tpu_sc as plsc

D_MODEL = 64
D_PAD = 128
NUM_CORES = 2
NUM_SUBCORES = 16
NUM_WORKERS = NUM_CORES * NUM_SUBCORES
CHUNK = 160
LANES = 16
NBUF = 2


def _sc_body(x_hbm, ts_hbm, tab_hbm, out_hbm, idx0, idx1, rows_v, x_v, res_v,
             gsem0, gsem1, xsem0, xsem1, osem0, osem1):
    idx_v = (idx0, idx1)
    gsem = (gsem0, gsem1)
    xsem = (xsem0, xsem1)
    osem = (osem0, osem1)
    n_tok = x_hbm.shape[0]
    per_w = n_tok // NUM_WORKERS
    n_chunks = per_w // CHUNK
    wid = lax.axis_index("s") * NUM_CORES + lax.axis_index("c")
    base_w = wid * per_w

    def issue(ci, b):
        base = base_w + ci * CHUNK
        pltpu.sync_copy(ts_hbm.at[pl.ds(base, CHUNK)], idx_v[b])
        pltpu.async_copy(tab_hbm.at[idx_v[b]], rows_v.at[b], gsem[b])
        pltpu.async_copy(x_hbm.at[pl.ds(base, CHUNK)], x_v.at[b], xsem[b])

    for b in range(NBUF):
        issue(b, b)

    @pl.loop(0, n_chunks, step=NBUF)
    def _(g):
        for b in range(NBUF):
            ci = g + b
            pltpu.make_async_copy(
                tab_hbm.at[idx_v[b]], rows_v.at[b], gsem[b]).wait()
            pltpu.make_async_copy(
                x_hbm.at[pl.ds(0, CHUNK)], x_v.at[b], xsem[b]).wait()

            @pl.when(ci >= NBUF)
            def _():
                pltpu.make_async_copy(
                    res_v.at[b], out_hbm.at[pl.ds(0, CHUNK)], osem[b]).wait()

            @plsc.parallel_loop(0, CHUNK, unroll=8)
            def _(i):
                for k in range(D_MODEL // LANES):
                    sl = pl.ds(k * LANES, LANES)
                    res_v[b, i, sl] = x_v[b, i, sl] + rows_v[b, i, sl]

            base = base_w + ci * CHUNK
            pltpu.async_copy(res_v.at[b], out_hbm.at[pl.ds(base, CHUNK)],
                             osem[b])

            @pl.when(ci + NBUF < n_chunks)
            def _():
                issue(ci + NBUF, b)

    for b in range(NBUF):
        pltpu.make_async_copy(
            res_v.at[b], out_hbm.at[pl.ds(0, CHUNK)], osem[b]).wait()


@jax.jit
def _sc_call(xf, ts, table_padded):
    n_tok = xf.shape[0]
    mesh = plsc.VectorSubcoreMesh(core_axis_name="c", subcore_axis_name="s")
    f = pl.kernel(
        _sc_body,
        mesh=mesh,
        out_type=jax.ShapeDtypeStruct((n_tok, D_MODEL), jnp.float32),
        scratch_types=[
            pltpu.VMEM((CHUNK,), jnp.int32),
            pltpu.VMEM((CHUNK,), jnp.int32),
            pltpu.VMEM((NBUF, CHUNK, D_PAD), jnp.float32),
            pltpu.VMEM((NBUF, CHUNK, D_MODEL), jnp.float32),
            pltpu.VMEM((NBUF, CHUNK, D_MODEL), jnp.float32),
            pltpu.SemaphoreType.DMA,
            pltpu.SemaphoreType.DMA,
            pltpu.SemaphoreType.DMA,
            pltpu.SemaphoreType.DMA,
            pltpu.SemaphoreType.DMA,
            pltpu.SemaphoreType.DMA,
        ],
    )
    return f(xf, ts, table_padded)


def kernel(x, timestamps, time_embedding):
    b, s, d = x.shape
    xf = x.reshape(b * s, d)
    ts = timestamps.reshape(-1).astype(jnp.int32)
    tab = jnp.pad(time_embedding, ((0, 0), (0, D_PAD - D_MODEL)))
    out = _sc_call(xf, ts, tab)
    return out.reshape(b, s, d)
